# Initial kernel scaffold; baseline (speedup 1.0000x reference)
import jax
import jax.numpy as jnp
from jax.experimental import pallas as pl


def kernel(gt_labels, gt_boxes, label_embed_weight):
    raise NotImplementedError("write your pallas kernel here")


# speedup vs baseline: 3.2881x; 3.2881x over previous
import jax
import jax.numpy as jnp
from jax.experimental import pallas as pl

_B = 16
_G = 100
_NUM_QUERIES = 900
_NUM_CLASSES = 80
_D = 1024
_GROUPS = 10
_LABEL_NOISE_PROB = 0.2
_BOX_NOISE_SCALE = 0.4
_Q = _G * _GROUPS
_N = _B * _G * _GROUPS
_TGT = _Q + _NUM_QUERIES


def _label_queries_body(p_ref, new_ref, lab_ref, e_ref, out_ref):
    p = p_ref[0]
    new = new_ref[0]
    lab = lab_ref[0]
    sel = jnp.where(p < _LABEL_NOISE_PROB, new, lab)
    iota = jax.lax.broadcasted_iota(jnp.int32, (_Q, _NUM_CLASSES), 1)
    onehot = (sel == iota).astype(jnp.float32)
    out_ref[0] = jnp.dot(onehot, e_ref[...],
                         preferred_element_type=jnp.float32)


def _box_body(boxes_ref, noise_ref, out_ref):
    b = boxes_ref[...]
    n = noise_ref[...]
    wh = b[:, 2:4]
    diff = jnp.concatenate([wh * 0.5, wh], axis=1)
    x = jnp.clip(b + n * diff * _BOX_NOISE_SCALE, 0.0, 1.0)
    x1 = jnp.maximum(x, 1e-5)
    x2 = jnp.maximum(1.0 - x, 1e-5)
    out_ref[...] = jnp.log(x1) - jnp.log(x2)


def _mask_body(out_ref):
    ii = jax.lax.broadcasted_iota(jnp.int32, (_TGT, _TGT), 0)
    jj = jax.lax.broadcasted_iota(jnp.int32, (_TGT, _TGT), 1)
    gi = (ii * 5243) >> 19
    gj = (jj * 5243) >> 19
    out_ref[...] = (jj < _Q) & ((ii >= _Q) | (gi != gj))


def kernel(gt_labels, gt_boxes, label_embed_weight):
    nk = jax.random.key(7)
    kp, kl, kb = jax.random.split(nk, 3)
    p = jax.random.uniform(kp, (_N,))
    new_labels = jax.random.randint(kl, (_N,), 0, _NUM_CLASSES, jnp.int32)
    noise = jax.random.uniform(kb, (_N, 4)) * 2.0 - 1.0
    p_d = p.reshape(_GROUPS, _B, _G).transpose(1, 0, 2).reshape(_B, _Q, 1)
    new_d = new_labels.reshape(_GROUPS, _B, _G).transpose(1, 0, 2).reshape(_B, _Q, 1)
    noise_d = noise.reshape(_GROUPS, _B, _G, 4).transpose(1, 0, 2, 3).reshape(_N, 4)
    lab_d = jnp.broadcast_to(gt_labels[:, None, :], (_B, _GROUPS, _G)).reshape(_B, _Q, 1)
    boxes_d = jnp.broadcast_to(gt_boxes[:, None], (_B, _GROUPS, _G, 4)).reshape(_N, 4)

    noised_label_queries = pl.pallas_call(
        _label_queries_body,
        grid=(_B,),
        in_specs=[
            pl.BlockSpec((1, _Q, 1), lambda b: (b, 0, 0)),
            pl.BlockSpec((1, _Q, 1), lambda b: (b, 0, 0)),
            pl.BlockSpec((1, _Q, 1), lambda b: (b, 0, 0)),
            pl.BlockSpec((_NUM_CLASSES, _D), lambda b: (0, 0)),
        ],
        out_specs=pl.BlockSpec((1, _Q, _D), lambda b: (b, 0, 0)),
        out_shape=jax.ShapeDtypeStruct((_B, _Q, _D), jnp.float32),
    )(p_d, new_d, lab_d, label_embed_weight)

    rows = 2000
    noised_box_queries = pl.pallas_call(
        _box_body,
        grid=(_N // rows,),
        in_specs=[
            pl.BlockSpec((rows, 4), lambda i: (i, 0)),
            pl.BlockSpec((rows, 4), lambda i: (i, 0)),
        ],
        out_specs=pl.BlockSpec((rows, 4), lambda i: (i, 0)),
        out_shape=jax.ShapeDtypeStruct((_N, 4), jnp.float32),
    )(boxes_d, noise_d).reshape(_B, _Q, 4)

    attn_mask = pl.pallas_call(
        _mask_body,
        out_shape=jax.ShapeDtypeStruct((_TGT, _TGT), jnp.bool_),
    )()

    return noised_label_queries, noised_box_queries, attn_mask


# speedup vs baseline: 3.4402x; 1.0462x over previous
import jax
import jax.numpy as jnp
from jax.experimental import pallas as pl

_B = 16
_G = 100
_NUM_QUERIES = 900
_NUM_CLASSES = 80
_D = 1024
_GROUPS = 10
_LABEL_NOISE_PROB = 0.2
_BOX_NOISE_SCALE = 0.4
_Q = _G * _GROUPS
_N = _B * _G * _GROUPS
_TGT = _Q + _NUM_QUERIES


_MROWS = 120


def _fused_body(p_ref, new_ref, lab_ref, e_ref, boxes_ref, noise_ref,
                lq_ref, bq_ref, mask_ref):
    p = p_ref[0]
    new = new_ref[0]
    lab = lab_ref[0]
    sel = jnp.where(p < _LABEL_NOISE_PROB, new, lab)
    iota = jax.lax.broadcasted_iota(jnp.int32, (_Q, _NUM_CLASSES), 1)
    onehot = (sel == iota).astype(jnp.float32)
    lq_ref[0] = jnp.dot(onehot, e_ref[...],
                        preferred_element_type=jnp.float32)

    b = boxes_ref[0]
    n = noise_ref[0]
    wh = b[:, 2:4]
    diff = jnp.concatenate([wh * 0.5, wh], axis=1)
    x = jnp.clip(b + n * diff * _BOX_NOISE_SCALE, 0.0, 1.0)
    x1 = jnp.maximum(x, 1e-5)
    x2 = jnp.maximum(1.0 - x, 1e-5)
    bq_ref[0] = jnp.log(x1) - jnp.log(x2)

    base = pl.program_id(0) * _MROWS
    ii = jax.lax.broadcasted_iota(jnp.int32, (_MROWS, _TGT), 0) + base
    jj = jax.lax.broadcasted_iota(jnp.int32, (_MROWS, _TGT), 1)
    gi = (ii * 5243) >> 19
    gj = (jj * 5243) >> 19
    mask_ref[...] = (jj < _Q) & ((ii >= _Q) | (gi != gj))


def kernel(gt_labels, gt_boxes, label_embed_weight):
    nk = jax.random.key(7)
    kp, kl, kb = jax.random.split(nk, 3)
    p = jax.random.uniform(kp, (_N,))
    new_labels = jax.random.randint(kl, (_N,), 0, _NUM_CLASSES, jnp.int32)
    noise = jax.random.uniform(kb, (_N, 4)) * 2.0 - 1.0
    p_d = p.reshape(_GROUPS, _B, _G).transpose(1, 0, 2).reshape(_B, _Q, 1)
    new_d = new_labels.reshape(_GROUPS, _B, _G).transpose(1, 0, 2).reshape(_B, _Q, 1)
    noise_d = noise.reshape(_GROUPS, _B, _G, 4).transpose(1, 0, 2, 3).reshape(_N, 4)
    lab_d = jnp.broadcast_to(gt_labels[:, None, :], (_B, _GROUPS, _G)).reshape(_B, _Q, 1)
    boxes_d = jnp.broadcast_to(gt_boxes[:, None], (_B, _GROUPS, _G, 4)).reshape(_N, 4)

    noised_label_queries, noised_box_queries, mask_padded = pl.pallas_call(
        _fused_body,
        grid=(_B,),
        in_specs=[
            pl.BlockSpec((1, _Q, 1), lambda b: (b, 0, 0)),
            pl.BlockSpec((1, _Q, 1), lambda b: (b, 0, 0)),
            pl.BlockSpec((1, _Q, 1), lambda b: (b, 0, 0)),
            pl.BlockSpec((_NUM_CLASSES, _D), lambda b: (0, 0)),
            pl.BlockSpec((1, _Q, 4), lambda b: (b, 0, 0)),
            pl.BlockSpec((1, _Q, 4), lambda b: (b, 0, 0)),
        ],
        out_specs=[
            pl.BlockSpec((1, _Q, _D), lambda b: (b, 0, 0)),
            pl.BlockSpec((1, _Q, 4), lambda b: (b, 0, 0)),
            pl.BlockSpec((_MROWS, _TGT), lambda b: (b, 0)),
        ],
        out_shape=[
            jax.ShapeDtypeStruct((_B, _Q, _D), jnp.float32),
            jax.ShapeDtypeStruct((_B, _Q, 4), jnp.float32),
            jax.ShapeDtypeStruct((_TGT, _TGT), jnp.bool_),
        ],
    )(p_d, new_d, lab_d, label_embed_weight,
      boxes_d.reshape(_B, _Q, 4), noise_d.reshape(_B, _Q, 4))

    return noised_label_queries, noised_box_queries, mask_padded
